# baseline (speedup 1.0000x reference)
import functools

import jax
import jax.numpy as jnp
from jax import lax
from jax.experimental import pallas as pl
from jax.experimental.pallas import tpu as pltpu
from jax.experimental.pallas import tpu_sc as plsc

MAX_POS = 8192
EMBED_DIM = 1024
BATCH = 4 * 4096

NUM_CORES = 2
NUM_SUBCORES = 16
NUM_WORKERS = NUM_CORES * NUM_SUBCORES
ROWS_PER_WORKER = BATCH // NUM_WORKERS
CHUNK = 8
NUM_CHUNKS = ROWS_PER_WORKER // CHUNK
NBUF = 14


def _make_gather():
    mesh = plsc.VectorSubcoreMesh(core_axis_name="c", subcore_axis_name="s")

    @functools.partial(
        pl.kernel,
        mesh=mesh,
        out_type=jax.ShapeDtypeStruct((BATCH, EMBED_DIM), jnp.float32),
        scratch_types=[
            pltpu.VMEM((ROWS_PER_WORKER,), jnp.int32),
            pltpu.VMEM((NBUF, CHUNK, EMBED_DIM), jnp.float32),
            pltpu.SemaphoreType.DMA((NBUF,)),
            pltpu.SemaphoreType.DMA((NBUF,)),
        ],
    )
    def gather_kernel(x_hbm, table_hbm, out_hbm, idx_v, rows_v, gsem, ssem):
        wid = lax.axis_index("s") * NUM_CORES + lax.axis_index("c")
        base = wid * ROWS_PER_WORKER
        pltpu.sync_copy(x_hbm.at[pl.ds(base, ROWS_PER_WORKER)], idx_v)

        def gather(k):
            b = k % NBUF
            return pltpu.async_copy(
                table_hbm.at[idx_v.at[pl.ds(k * CHUNK, CHUNK)]],
                rows_v.at[b],
                gsem.at[b],
            )

        def store(k):
            b = k % NBUF
            return pltpu.async_copy(
                rows_v.at[b],
                out_hbm.at[pl.ds(base + k * CHUNK, CHUNK)],
                ssem.at[b],
            )

        A = NBUF - 2
        g_descs = [None] * NUM_CHUNKS
        s_descs = [None] * NUM_CHUNKS
        for j in range(A):
            g_descs[j] = gather(j)
        for k in range(NUM_CHUNKS):
            j = k + A
            if j < NUM_CHUNKS:
                if j - NBUF >= 0:
                    s_descs[j - NBUF].wait()
                g_descs[j] = gather(j)
            g_descs[k].wait()
            s_descs[k] = store(k)
        for k in range(max(0, NUM_CHUNKS - NBUF), NUM_CHUNKS):
            s_descs[k].wait()

    return gather_kernel


_gather = _make_gather()


@jax.jit
def kernel(x, embeddings):
    flat = x.reshape(BATCH)
    out = _gather(flat, embeddings)
    return out.reshape(x.shape + (EMBED_DIM,))


# speedup vs baseline: 1.0090x; 1.0090x over previous
import functools

import jax
import jax.numpy as jnp
from jax import lax
from jax.experimental import pallas as pl
from jax.experimental.pallas import tpu as pltpu
from jax.experimental.pallas import tpu_sc as plsc

MAX_POS = 8192
EMBED_DIM = 1024
BATCH = 4 * 4096

NUM_CORES = 2
NUM_SUBCORES = 16
NUM_WORKERS = NUM_CORES * NUM_SUBCORES
ROWS_PER_WORKER = BATCH // NUM_WORKERS
CHUNK = 16
NUM_CHUNKS = ROWS_PER_WORKER // CHUNK
NBUF = 6
LANES = 16


def _make_gather():
    mesh = plsc.VectorSubcoreMesh(core_axis_name="c", subcore_axis_name="s")

    @functools.partial(
        pl.kernel,
        mesh=mesh,
        out_type=jax.ShapeDtypeStruct((BATCH, EMBED_DIM), jnp.float32),
        scratch_types=[
            pltpu.VMEM((ROWS_PER_WORKER,), jnp.int32),
            pltpu.VMEM((NUM_CHUNKS, CHUNK), jnp.int32),
            pltpu.VMEM((NBUF, CHUNK, EMBED_DIM), jnp.float32),
            pltpu.SemaphoreType.DMA((NBUF,)),
            pltpu.SemaphoreType.DMA((NBUF,)),
        ],
    )
    def gather_kernel(x_hbm, table_hbm, out_hbm, idx_v, dest_v, rows_v,
                      gsem, ssem):
        wid = lax.axis_index("s") * NUM_CORES + lax.axis_index("c")
        base = wid * ROWS_PER_WORKER
        pltpu.sync_copy(x_hbm.at[pl.ds(base, ROWS_PER_WORKER)], idx_v)

        lane = lax.iota(jnp.int32, LANES)
        for k in range(NUM_CHUNKS):
            dest_v[k, :] = lane + (base + k * CHUNK)

        def gather(k):
            b = k % NBUF
            return pltpu.async_copy(
                table_hbm.at[idx_v.at[pl.ds(k * CHUNK, CHUNK)]],
                rows_v.at[b],
                gsem.at[b],
            )

        def store(k):
            b = k % NBUF
            return pltpu.async_copy(
                rows_v.at[b],
                out_hbm.at[dest_v.at[k]],
                ssem.at[b],
            )

        A = NBUF - 2
        g_descs = [None] * NUM_CHUNKS
        s_descs = [None] * NUM_CHUNKS
        for j in range(A):
            g_descs[j] = gather(j)
        for k in range(NUM_CHUNKS):
            j = k + A
            if j < NUM_CHUNKS:
                if j - NBUF >= 0:
                    s_descs[j - NBUF].wait()
                g_descs[j] = gather(j)
            g_descs[k].wait()
            s_descs[k] = store(k)
        for k in range(max(0, NUM_CHUNKS - NBUF), NUM_CHUNKS):
            s_descs[k].wait()

    return gather_kernel


_gather = _make_gather()


@jax.jit
def kernel(x, embeddings):
    flat = x.reshape(BATCH)
    out = _gather(flat, embeddings)
    return out.reshape(x.shape + (EMBED_DIM,))


# speedup vs baseline: 1.0260x; 1.0168x over previous
import functools

import jax
import jax.numpy as jnp
from jax import lax
from jax.experimental import pallas as pl
from jax.experimental.pallas import tpu as pltpu
from jax.experimental.pallas import tpu_sc as plsc

MAX_POS = 8192
EMBED_DIM = 1024
BATCH = 4 * 4096

NUM_CORES = 2
NUM_SUBCORES = 16
NUM_WORKERS = NUM_CORES * NUM_SUBCORES
ROWS_PER_WORKER = BATCH // NUM_WORKERS
CHUNK = 16
NUM_CHUNKS = ROWS_PER_WORKER // CHUNK
NBUF = 7
LANES = 16


def _make_gather():
    mesh = plsc.VectorSubcoreMesh(core_axis_name="c", subcore_axis_name="s")

    @functools.partial(
        pl.kernel,
        mesh=mesh,
        out_type=jax.ShapeDtypeStruct((BATCH, EMBED_DIM), jnp.float32),
        scratch_types=[
            pltpu.VMEM((ROWS_PER_WORKER,), jnp.int32),
            pltpu.VMEM((NBUF, CHUNK, EMBED_DIM), jnp.float32),
            pltpu.SemaphoreType.DMA((NBUF,)),
            pltpu.SemaphoreType.DMA((NBUF,)),
        ],
    )
    def gather_kernel(x_hbm, table_hbm, out_hbm, idx_v, rows_v, gsem, ssem):
        wid = lax.axis_index("s") * NUM_CORES + lax.axis_index("c")
        base = wid * ROWS_PER_WORKER
        pltpu.sync_copy(x_hbm.at[pl.ds(base, ROWS_PER_WORKER)], idx_v)

        def gather(k):
            b = k % NBUF
            return pltpu.async_copy(
                table_hbm.at[idx_v.at[pl.ds(k * CHUNK, CHUNK)]],
                rows_v.at[b],
                gsem.at[b],
            )

        def store(k):
            b = k % NBUF
            return pltpu.async_copy(
                rows_v.at[b],
                out_hbm.at[pl.ds(base + k * CHUNK, CHUNK)],
                ssem.at[b],
            )

        A = NBUF - 2
        g_descs = [None] * NUM_CHUNKS
        s_descs = [None] * NUM_CHUNKS
        for j in range(A):
            g_descs[j] = gather(j)
        for k in range(NUM_CHUNKS):
            j = k + A
            if j < NUM_CHUNKS:
                if j - NBUF >= 0:
                    s_descs[j - NBUF].wait()
                g_descs[j] = gather(j)
            g_descs[k].wait()
            s_descs[k] = store(k)
        for k in range(max(0, NUM_CHUNKS - NBUF), NUM_CHUNKS):
            s_descs[k].wait()

    return gather_kernel


_gather = _make_gather()


@jax.jit
def kernel(x, embeddings):
    flat = x.reshape(BATCH)
    out = _gather(flat, embeddings)
    return out.reshape(x.shape + (EMBED_DIM,))


# speedup vs baseline: 1.0333x; 1.0071x over previous
import functools

import jax
import jax.numpy as jnp
from jax import lax
from jax.experimental import pallas as pl
from jax.experimental.pallas import tpu as pltpu
from jax.experimental.pallas import tpu_sc as plsc

MAX_POS = 8192
EMBED_DIM = 1024
BATCH = 4 * 4096

NUM_CORES = 2
NUM_SUBCORES = 16
NUM_WORKERS = NUM_CORES * NUM_SUBCORES
ROWS_PER_WORKER = BATCH // NUM_WORKERS
CHUNK = 16
NUM_CHUNKS = ROWS_PER_WORKER // CHUNK
NBUF = 6


def _make_gather():
    mesh = plsc.VectorSubcoreMesh(core_axis_name="c", subcore_axis_name="s")

    @functools.partial(
        pl.kernel,
        mesh=mesh,
        out_type=jax.ShapeDtypeStruct((BATCH, EMBED_DIM), jnp.float32),
        scratch_types=[
            pltpu.VMEM((ROWS_PER_WORKER,), jnp.int32),
            pltpu.VMEM((NBUF, CHUNK, EMBED_DIM), jnp.float32),
            pltpu.SemaphoreType.DMA((NBUF,)),
            pltpu.SemaphoreType.DMA((NBUF,)),
        ],
    )
    def gather_kernel(x_hbm, table_hbm, out_hbm, idx_v, rows_v, gsem, ssem):
        wid = lax.axis_index("s") * NUM_CORES + lax.axis_index("c")
        base = wid * ROWS_PER_WORKER
        pltpu.sync_copy(x_hbm.at[pl.ds(base, ROWS_PER_WORKER)], idx_v)

        def gather(k):
            b = k % NBUF
            return pltpu.async_copy(
                table_hbm.at[idx_v.at[pl.ds(k * CHUNK, CHUNK)]],
                rows_v.at[b],
                gsem.at[b],
            )

        def store(k):
            b = k % NBUF
            return pltpu.async_copy(
                rows_v.at[b],
                out_hbm.at[pl.ds(base + k * CHUNK, CHUNK)],
                ssem.at[b],
            )

        A = NBUF - 2
        g_descs = [None] * NUM_CHUNKS
        s_descs = [None] * NUM_CHUNKS
        for j in range(A):
            g_descs[j] = gather(j)
        for k in range(NUM_CHUNKS):
            j = k + A
            if j < NUM_CHUNKS:
                if j - NBUF >= 0:
                    s_descs[j - NBUF].wait()
                g_descs[j] = gather(j)
            g_descs[k].wait()
            s_descs[k] = store(k)
        for k in range(max(0, NUM_CHUNKS - NBUF), NUM_CHUNKS):
            s_descs[k].wait()

    return gather_kernel


_gather = _make_gather()


@jax.jit
def kernel(x, embeddings):
    flat = x.reshape(BATCH)
    out = _gather(flat, embeddings)
    return out.reshape(x.shape + (EMBED_DIM,))
